# baseline (speedup 1.0000x reference)
import functools

import jax
import jax.numpy as jnp
from jax import lax
from jax.experimental import pallas as pl
from jax.experimental.pallas import tpu as pltpu
from jax.experimental.pallas import tpu_sc as plsc

N = 10000
E = 160000
D = 256
H = 128

NC = 2
NS = 16
B = 128
NBLK = 80
SBLK = 40
EPT = NBLK * B
E_PAD = NS * EPT
NACC = 10240
ZPT = NACC // NS

_mesh = plsc.VectorSubcoreMesh(
    core_axis_name="c", subcore_axis_name="s", num_cores=NC, num_subcores=NS)


def _deg_body(dst_hbm, z1_hbm, ones1_hbm, deg_out, dacc, dst_v, ones_v):
  c = lax.axis_index("c")
  s = lax.axis_index("s")
  pltpu.sync_copy(z1_hbm.at[pl.ds(s * ZPT, ZPT)], dacc.at[pl.ds(s * ZPT, ZPT)])
  pltpu.sync_copy(ones1_hbm, ones_v)
  pltpu.sync_copy(dst_hbm.at[c * NS + s], dst_v)
  plsc.subcore_barrier()

  def blk(i, carry):
    pltpu.sync_copy(ones_v, dacc.at[dst_v.at[i]], add=True)
    return carry

  lax.fori_loop(0, NBLK, blk, 0)
  plsc.subcore_barrier()
  pltpu.sync_copy(dacc.at[pl.ds(s * ZPT, ZPT)],
                  deg_out.at[pl.ds(c * NACC + s * ZPT, ZPT)])


_deg_kernel = functools.partial(
    pl.kernel, _deg_body,
    out_type=jax.ShapeDtypeStruct((NC * NACC,), jnp.float32),
    mesh=_mesh,
    scratch_types=[
        pltpu.VMEM_SHARED((NACC,), jnp.float32),
        pltpu.VMEM((NBLK, B), jnp.int32),
        pltpu.VMEM((B,), jnp.float32),
    ],
)()


def _scat_body(ysp_hbm, ytr_hbm, gidx_hbm, dst_hbm, z_hbm, s_out,
               acc, gidx_v, dst_v, rows_a, rows_b, sem_a, sem_b):
  c = lax.axis_index("c")
  s = lax.axis_index("s")
  for set_ in range(2):
    table = ysp_hbm if set_ == 0 else ytr_hbm
    pltpu.sync_copy(z_hbm.at[pl.ds(s * ZPT, ZPT)], acc.at[pl.ds(s * ZPT, ZPT)])
    plsc.subcore_barrier()

    for stage in range(NBLK // SBLK):
      pltpu.sync_copy(
          gidx_hbm.at[((set_ * NC + c) * NS + s) * (NBLK // SBLK) + stage],
          gidx_v)
      pltpu.sync_copy(
          dst_hbm.at[(set_ * NS + s) * (NBLK // SBLK) + stage], dst_v)

      pltpu.async_copy(table.at[gidx_v.at[0]], rows_a, sem_a)

      def blk(j, carry):
        i0 = 2 * j
        pltpu.make_async_copy(table.at[gidx_v.at[i0]], rows_a, sem_a).wait()
        pltpu.async_copy(table.at[gidx_v.at[i0 + 1]], rows_b, sem_b)
        pltpu.sync_copy(rows_a, acc.at[dst_v.at[i0]], add=True)
        pltpu.make_async_copy(table.at[gidx_v.at[i0 + 1]], rows_b, sem_b).wait()

        @pl.when(j < SBLK // 2 - 1)
        def _():
          pltpu.async_copy(table.at[gidx_v.at[i0 + 2]], rows_a, sem_a)

        pltpu.sync_copy(rows_b, acc.at[dst_v.at[i0 + 1]], add=True)
        return carry

      lax.fori_loop(0, SBLK // 2, blk, 0)
    plsc.subcore_barrier()
    pltpu.sync_copy(
        acc.at[pl.ds(s * ZPT, ZPT)],
        s_out.at[pl.ds((set_ * NC + c) * NACC + s * ZPT, ZPT)])
    plsc.subcore_barrier()


_scat_kernel = functools.partial(
    pl.kernel, _scat_body,
    out_type=jax.ShapeDtypeStruct((2 * NC * NACC, H), jnp.float32),
    mesh=_mesh,
    scratch_types=[
        pltpu.VMEM_SHARED((NACC, H), jnp.float32),
        pltpu.VMEM((SBLK, B), jnp.int32),
        pltpu.VMEM((SBLK, B), jnp.int32),
        pltpu.VMEM((B, H), jnp.float32),
        pltpu.VMEM((B, H), jnp.float32),
        pltpu.SemaphoreType.DMA,
        pltpu.SemaphoreType.DMA,
    ],
)()


BN = 2000


def _prep_body(x_ref, w_ref, dsp_ref, dtr_ref, ysp_ref, ytr_ref):
  xw = jnp.dot(x_ref[...], w_ref[...], preferred_element_type=jnp.float32)
  dis_sp = lax.rsqrt(dsp_ref[...][:, 0:1] + 1.0)
  dis_tr = lax.rsqrt(dtr_ref[...][:, 0:1] + 1.0)
  ysp_ref[...] = xw[:, :D] * dis_sp
  ytr_ref[...] = xw[:, D:] * dis_tr


def _final_body(ysp_ref, ytr_ref, s_ref, dsp_ref, dtr_ref, bsp_ref, btr_ref,
                wtop_ref, wbot_ref, bf_ref, out_ref):
  dis_sp = lax.rsqrt(dsp_ref[...][:, 0:1] + 1.0)
  dis_tr = lax.rsqrt(dtr_ref[...][:, 0:1] + 1.0)
  s4 = s_ref[...]
  s_sp = jnp.concatenate([s4[0, 0], s4[0, 1]], axis=-1)
  s_tr = jnp.concatenate([s4[1, 0], s4[1, 1]], axis=-1)
  h_s = jnp.maximum((ysp_ref[...] + s_sp) * dis_sp + bsp_ref[...], 0.0)
  h_t = jnp.maximum((ytr_ref[...] + s_tr) * dis_tr + btr_ref[...], 0.0)
  acc = jnp.dot(h_s, wtop_ref[...], preferred_element_type=jnp.float32)
  acc += jnp.dot(h_t, wbot_ref[...], preferred_element_type=jnp.float32)
  out_ref[...] = jnp.maximum(acc + bf_ref[...], 0.0)


def _pad_set(idx, fill):
  return jnp.concatenate(
      [idx, jnp.full((E_PAD - E,), fill, dtype=jnp.int32)])


def kernel(x, sp_ei, tr_ei, W_spa, b_spa, W_tra, b_tra, W_fuse, b_fuse):
  sp_src = sp_ei[0].astype(jnp.int32)
  sp_dst = sp_ei[1].astype(jnp.int32)
  tr_src = tr_ei[0].astype(jnp.int32)
  tr_dst = tr_ei[1].astype(jnp.int32)

  sp_src, sp_dst = lax.sort([sp_src, sp_dst], num_keys=1)
  tr_src, tr_dst = lax.sort([tr_src, tr_dst], num_keys=1)

  dst_flat = jnp.stack([_pad_set(sp_dst, N), _pad_set(tr_dst, N)])
  dst_deg = dst_flat.reshape(2 * NS, NBLK, B)
  dst_all = dst_flat.reshape(2 * NS * (NBLK // SBLK), SBLK, B)
  gidx = jnp.stack([
      jnp.stack([_pad_set(2 * sp_src, 0), _pad_set(2 * sp_src + 1, 0)]),
      jnp.stack([_pad_set(2 * tr_src, 0), _pad_set(2 * tr_src + 1, 0)]),
  ]).reshape(2 * NC * NS * (NBLK // SBLK), SBLK, B)

  z1 = jnp.zeros((NACC,), jnp.float32)
  z128 = jnp.zeros((NACC, H), jnp.float32)
  ones1 = jnp.ones((B,), jnp.float32)

  deg_flat = _deg_kernel(dst_deg, z1, ones1)
  deg_sp = jnp.broadcast_to(deg_flat[:N, None], (N, 16))
  deg_tr = jnp.broadcast_to(deg_flat[NACC:NACC + N, None], (N, 16))

  W_cat = jnp.concatenate([W_spa, W_tra], axis=1)
  grid = N // BN
  y_sp, y_tr = pl.pallas_call(
      _prep_body,
      grid=(grid,),
      in_specs=[
          pl.BlockSpec((BN, D), lambda i: (i, 0)),
          pl.BlockSpec((D, 2 * D), lambda i: (0, 0)),
          pl.BlockSpec((BN, 16), lambda i: (i, 0)),
          pl.BlockSpec((BN, 16), lambda i: (i, 0)),
      ],
      out_specs=[
          pl.BlockSpec((BN, D), lambda i: (i, 0)),
          pl.BlockSpec((BN, D), lambda i: (i, 0)),
      ],
      out_shape=[
          jax.ShapeDtypeStruct((N, D), jnp.float32),
          jax.ShapeDtypeStruct((N, D), jnp.float32),
      ],
  )(x, W_cat, deg_sp, deg_tr)

  s_flat = _scat_kernel(
      y_sp.reshape(2 * N, H), y_tr.reshape(2 * N, H), gidx, dst_all, z128)
  s4 = s_flat.reshape(2, NC, NACC, H)[:, :, :N, :]

  out = pl.pallas_call(
      _final_body,
      grid=(grid,),
      in_specs=[
          pl.BlockSpec((BN, D), lambda i: (i, 0)),
          pl.BlockSpec((BN, D), lambda i: (i, 0)),
          pl.BlockSpec((2, NC, BN, H), lambda i: (0, 0, i, 0)),
          pl.BlockSpec((BN, 16), lambda i: (i, 0)),
          pl.BlockSpec((BN, 16), lambda i: (i, 0)),
          pl.BlockSpec((1, D), lambda i: (0, 0)),
          pl.BlockSpec((1, D), lambda i: (0, 0)),
          pl.BlockSpec((D, D), lambda i: (0, 0)),
          pl.BlockSpec((D, D), lambda i: (0, 0)),
          pl.BlockSpec((1, D), lambda i: (0, 0)),
      ],
      out_specs=pl.BlockSpec((BN, D), lambda i: (i, 0)),
      out_shape=jax.ShapeDtypeStruct((N, D), jnp.float32),
  )(y_sp, y_tr, s4, deg_sp, deg_tr,
    b_spa.reshape(1, D), b_tra.reshape(1, D),
    W_fuse[:D], W_fuse[D:], b_fuse.reshape(1, D))
  return out


# speedup vs baseline: 1.5509x; 1.5509x over previous
import functools

import jax
import jax.numpy as jnp
from jax import lax
from jax.experimental import pallas as pl
from jax.experimental.pallas import tpu as pltpu
from jax.experimental.pallas import tpu_sc as plsc

N = 10000
E = 160000
D = 256
H = 128

NC = 2
NS = 16
B = 128
NBLK = 80
SBLK = 40
EPT = NBLK * B
E_PAD = NS * EPT
NACC = 10240
ZPT = NACC // NS

_mesh = plsc.VectorSubcoreMesh(
    core_axis_name="c", subcore_axis_name="s", num_cores=NC, num_subcores=NS)


def _deg_body(dst_hbm, z1_hbm, ones1_hbm, deg_out, dacc, dst_v, ones_v):
  c = lax.axis_index("c")
  s = lax.axis_index("s")
  pltpu.sync_copy(z1_hbm.at[pl.ds(s * ZPT, ZPT)], dacc.at[pl.ds(s * ZPT, ZPT)])
  pltpu.sync_copy(ones1_hbm, ones_v)
  pltpu.sync_copy(dst_hbm.at[c * NS + s], dst_v)
  plsc.subcore_barrier()

  def blk(i, carry):
    pltpu.sync_copy(ones_v, dacc.at[dst_v.at[i]], add=True)
    return carry

  lax.fori_loop(0, NBLK, blk, 0)
  plsc.subcore_barrier()
  pltpu.sync_copy(dacc.at[pl.ds(s * ZPT, ZPT)],
                  deg_out.at[pl.ds(c * NACC + s * ZPT, ZPT)])


_deg_kernel = functools.partial(
    pl.kernel, _deg_body,
    out_type=jax.ShapeDtypeStruct((NC * NACC,), jnp.float32),
    mesh=_mesh,
    scratch_types=[
        pltpu.VMEM_SHARED((NACC,), jnp.float32),
        pltpu.VMEM((NBLK, B), jnp.int32),
        pltpu.VMEM((B,), jnp.float32),
    ],
)()


def _scat_body(ysp_hbm, ytr_hbm, gidx_hbm, dst_hbm, z_hbm, s_out,
               acc, gidx_v, dst_v, rows_a, rows_b, sem_a, sem_b):
  c = lax.axis_index("c")
  s = lax.axis_index("s")
  for set_ in range(2):
    table = ysp_hbm if set_ == 0 else ytr_hbm
    pltpu.sync_copy(z_hbm.at[pl.ds(s * ZPT, ZPT)], acc.at[pl.ds(s * ZPT, ZPT)])
    plsc.subcore_barrier()

    for stage in range(NBLK // SBLK):
      pltpu.sync_copy(
          gidx_hbm.at[((set_ * NC + c) * NS + s) * (NBLK // SBLK) + stage],
          gidx_v)
      pltpu.sync_copy(
          dst_hbm.at[(set_ * NS + s) * (NBLK // SBLK) + stage], dst_v)

      pltpu.async_copy(table.at[gidx_v.at[0]], rows_a, sem_a)

      def blk(j, carry):
        i0 = 2 * j
        pltpu.make_async_copy(table.at[gidx_v.at[i0]], rows_a, sem_a).wait()
        pltpu.async_copy(table.at[gidx_v.at[i0 + 1]], rows_b, sem_b)
        pltpu.sync_copy(rows_a, acc.at[dst_v.at[i0]], add=True)
        pltpu.make_async_copy(table.at[gidx_v.at[i0 + 1]], rows_b, sem_b).wait()

        @pl.when(j < SBLK // 2 - 1)
        def _():
          pltpu.async_copy(table.at[gidx_v.at[i0 + 2]], rows_a, sem_a)

        pltpu.sync_copy(rows_b, acc.at[dst_v.at[i0 + 1]], add=True)
        return carry

      lax.fori_loop(0, SBLK // 2, blk, 0)
    plsc.subcore_barrier()
    pltpu.sync_copy(
        acc.at[pl.ds(s * ZPT, ZPT)],
        s_out.at[pl.ds((set_ * NC + c) * NACC + s * ZPT, ZPT)])
    plsc.subcore_barrier()


_scat_kernel = functools.partial(
    pl.kernel, _scat_body,
    out_type=jax.ShapeDtypeStruct((2 * NC * NACC, H), jnp.float32),
    mesh=_mesh,
    scratch_types=[
        pltpu.VMEM_SHARED((NACC, H), jnp.float32),
        pltpu.VMEM((SBLK, B), jnp.int32),
        pltpu.VMEM((SBLK, B), jnp.int32),
        pltpu.VMEM((B, H), jnp.float32),
        pltpu.VMEM((B, H), jnp.float32),
        pltpu.SemaphoreType.DMA,
        pltpu.SemaphoreType.DMA,
    ],
)()


BN = 2000


def _prep_body(x_ref, w_ref, dsp_ref, dtr_ref, ysp_ref, ytr_ref):
  xw = jnp.dot(x_ref[...], w_ref[...], preferred_element_type=jnp.float32)
  dis_sp = lax.rsqrt(dsp_ref[...][:, 0:1] + 1.0)
  dis_tr = lax.rsqrt(dtr_ref[...][:, 0:1] + 1.0)
  ysp_ref[...] = xw[:, :D] * dis_sp
  ytr_ref[...] = xw[:, D:] * dis_tr


def _final_body(ysp_ref, ytr_ref, s_ref, dsp_ref, dtr_ref, bsp_ref, btr_ref,
                wtop_ref, wbot_ref, bf_ref, out_ref):
  dis_sp = lax.rsqrt(dsp_ref[...][:, 0:1] + 1.0)
  dis_tr = lax.rsqrt(dtr_ref[...][:, 0:1] + 1.0)
  s4 = s_ref[...]
  s_sp = jnp.concatenate([s4[0, 0], s4[0, 1]], axis=-1)
  s_tr = jnp.concatenate([s4[1, 0], s4[1, 1]], axis=-1)
  h_s = jnp.maximum((ysp_ref[...] + s_sp) * dis_sp + bsp_ref[...], 0.0)
  h_t = jnp.maximum((ytr_ref[...] + s_tr) * dis_tr + btr_ref[...], 0.0)
  acc = jnp.dot(h_s, wtop_ref[...], preferred_element_type=jnp.float32)
  acc += jnp.dot(h_t, wbot_ref[...], preferred_element_type=jnp.float32)
  out_ref[...] = jnp.maximum(acc + bf_ref[...], 0.0)


def _pad_set(idx, fill):
  return jnp.concatenate(
      [idx, jnp.full((E_PAD - E,), fill, dtype=jnp.int32)])


def kernel(x, sp_ei, tr_ei, W_spa, b_spa, W_tra, b_tra, W_fuse, b_fuse):
  sp_src = sp_ei[0].astype(jnp.int32)
  sp_dst = sp_ei[1].astype(jnp.int32)
  tr_src = tr_ei[0].astype(jnp.int32)
  tr_dst = tr_ei[1].astype(jnp.int32)

  dst_flat = jnp.stack([_pad_set(sp_dst, N), _pad_set(tr_dst, N)])
  dst_deg = dst_flat.reshape(2 * NS, NBLK, B)
  dst_all = dst_flat.reshape(2 * NS * (NBLK // SBLK), SBLK, B)
  gidx = jnp.stack([
      jnp.stack([_pad_set(2 * sp_src, 0), _pad_set(2 * sp_src + 1, 0)]),
      jnp.stack([_pad_set(2 * tr_src, 0), _pad_set(2 * tr_src + 1, 0)]),
  ]).reshape(2 * NC * NS * (NBLK // SBLK), SBLK, B)

  z1 = jnp.zeros((NACC,), jnp.float32)
  z128 = jnp.zeros((NACC, H), jnp.float32)
  ones1 = jnp.ones((B,), jnp.float32)

  deg_flat = _deg_kernel(dst_deg, z1, ones1)
  deg_sp = jnp.broadcast_to(deg_flat[:N, None], (N, 16))
  deg_tr = jnp.broadcast_to(deg_flat[NACC:NACC + N, None], (N, 16))

  W_cat = jnp.concatenate([W_spa, W_tra], axis=1)
  grid = N // BN
  y_sp, y_tr = pl.pallas_call(
      _prep_body,
      grid=(grid,),
      in_specs=[
          pl.BlockSpec((BN, D), lambda i: (i, 0)),
          pl.BlockSpec((D, 2 * D), lambda i: (0, 0)),
          pl.BlockSpec((BN, 16), lambda i: (i, 0)),
          pl.BlockSpec((BN, 16), lambda i: (i, 0)),
      ],
      out_specs=[
          pl.BlockSpec((BN, D), lambda i: (i, 0)),
          pl.BlockSpec((BN, D), lambda i: (i, 0)),
      ],
      out_shape=[
          jax.ShapeDtypeStruct((N, D), jnp.float32),
          jax.ShapeDtypeStruct((N, D), jnp.float32),
      ],
  )(x, W_cat, deg_sp, deg_tr)

  s_flat = _scat_kernel(
      y_sp.reshape(2 * N, H), y_tr.reshape(2 * N, H), gidx, dst_all, z128)
  s4 = s_flat.reshape(2, NC, NACC, H)[:, :, :N, :]

  out = pl.pallas_call(
      _final_body,
      grid=(grid,),
      in_specs=[
          pl.BlockSpec((BN, D), lambda i: (i, 0)),
          pl.BlockSpec((BN, D), lambda i: (i, 0)),
          pl.BlockSpec((2, NC, BN, H), lambda i: (0, 0, i, 0)),
          pl.BlockSpec((BN, 16), lambda i: (i, 0)),
          pl.BlockSpec((BN, 16), lambda i: (i, 0)),
          pl.BlockSpec((1, D), lambda i: (0, 0)),
          pl.BlockSpec((1, D), lambda i: (0, 0)),
          pl.BlockSpec((D, D), lambda i: (0, 0)),
          pl.BlockSpec((D, D), lambda i: (0, 0)),
          pl.BlockSpec((1, D), lambda i: (0, 0)),
      ],
      out_specs=pl.BlockSpec((BN, D), lambda i: (i, 0)),
      out_shape=jax.ShapeDtypeStruct((N, D), jnp.float32),
  )(y_sp, y_tr, s4, deg_sp, deg_tr,
    b_spa.reshape(1, D), b_tra.reshape(1, D),
    W_fuse[:D], W_fuse[D:], b_fuse.reshape(1, D))
  return out


# speedup vs baseline: 1.5743x; 1.0151x over previous
import functools

import jax
import jax.numpy as jnp
from jax import lax
from jax.experimental import pallas as pl
from jax.experimental.pallas import tpu as pltpu
from jax.experimental.pallas import tpu_sc as plsc

N = 10000
E = 160000
D = 256
H = 128

NC = 2
NS = 16
B = 128
NBLK = 80
SBLK = 40
EPT = NBLK * B
E_PAD = NS * EPT
NACC = 10240
ZPT = NACC // NS

_mesh = plsc.VectorSubcoreMesh(
    core_axis_name="c", subcore_axis_name="s", num_cores=NC, num_subcores=NS)


def _deg_body(dst_hbm, z1_hbm, ones1_hbm, deg_out, dacc, dst_v, ones_v):
  c = lax.axis_index("c")
  s = lax.axis_index("s")
  pltpu.sync_copy(z1_hbm.at[pl.ds(s * ZPT, ZPT)], dacc.at[pl.ds(s * ZPT, ZPT)])
  pltpu.sync_copy(ones1_hbm, ones_v)
  pltpu.sync_copy(dst_hbm.at[c * NS + s], dst_v)
  plsc.subcore_barrier()

  def blk(i, carry):
    pltpu.sync_copy(ones_v, dacc.at[dst_v.at[i]], add=True)
    return carry

  lax.fori_loop(0, NBLK, blk, 0)
  plsc.subcore_barrier()
  pltpu.sync_copy(dacc.at[pl.ds(s * ZPT, ZPT)],
                  deg_out.at[pl.ds(c * NACC + s * ZPT, ZPT)])


_deg_kernel = functools.partial(
    pl.kernel, _deg_body,
    out_type=jax.ShapeDtypeStruct((NC * NACC,), jnp.float32),
    mesh=_mesh,
    scratch_types=[
        pltpu.VMEM_SHARED((NACC,), jnp.float32),
        pltpu.VMEM((NBLK, B), jnp.int32),
        pltpu.VMEM((B,), jnp.float32),
    ],
)()


def _scat_body(ysp_hbm, ytr_hbm, gidx_hbm, dst_hbm, z_hbm, s_out,
               acc, gidx_v, dst_v, rows_a, rows_b, sem_a, sem_b):
  c = lax.axis_index("c")
  s = lax.axis_index("s")
  for set_ in range(2):
    table = ysp_hbm if set_ == 0 else ytr_hbm
    pltpu.sync_copy(z_hbm.at[pl.ds(s * ZPT, ZPT)], acc.at[pl.ds(s * ZPT, ZPT)])
    plsc.subcore_barrier()

    for stage in range(NBLK // SBLK):
      pltpu.sync_copy(
          gidx_hbm.at[((set_ * NC + c) * NS + s) * (NBLK // SBLK) + stage],
          gidx_v)
      pltpu.sync_copy(
          dst_hbm.at[(set_ * NS + s) * (NBLK // SBLK) + stage], dst_v)

      pltpu.async_copy(table.at[gidx_v.at[0]], rows_a, sem_a)

      def blk(j, carry):
        i0 = 2 * j
        pltpu.make_async_copy(table.at[gidx_v.at[i0]], rows_a, sem_a).wait()
        pltpu.async_copy(table.at[gidx_v.at[i0 + 1]], rows_b, sem_b)
        pltpu.sync_copy(rows_a, acc.at[dst_v.at[i0]], add=True)
        pltpu.make_async_copy(table.at[gidx_v.at[i0 + 1]], rows_b, sem_b).wait()

        @pl.when(j < SBLK // 2 - 1)
        def _():
          pltpu.async_copy(table.at[gidx_v.at[i0 + 2]], rows_a, sem_a)

        pltpu.sync_copy(rows_b, acc.at[dst_v.at[i0 + 1]], add=True)
        return carry

      lax.fori_loop(0, SBLK // 2, blk, 0)
    plsc.subcore_barrier()
    pltpu.sync_copy(
        acc.at[pl.ds(s * ZPT, ZPT)],
        s_out.at[pl.ds((set_ * NC + c) * NACC + s * ZPT, ZPT)])
    plsc.subcore_barrier()


_scat_kernel = functools.partial(
    pl.kernel, _scat_body,
    out_type=jax.ShapeDtypeStruct((2 * NC * NACC, H), jnp.float32),
    mesh=_mesh,
    scratch_types=[
        pltpu.VMEM_SHARED((NACC, H), jnp.float32),
        pltpu.VMEM((SBLK, B), jnp.int32),
        pltpu.VMEM((SBLK, B), jnp.int32),
        pltpu.VMEM((B, H), jnp.float32),
        pltpu.VMEM((B, H), jnp.float32),
        pltpu.SemaphoreType.DMA,
        pltpu.SemaphoreType.DMA,
    ],
)()


BN = 2000


def _prep_body(x_ref, w_ref, dsp_ref, dtr_ref, ysp_ref, ytr_ref):
  xw = jnp.dot(x_ref[...], w_ref[...], preferred_element_type=jnp.float32)
  dis_sp = lax.rsqrt(dsp_ref[...][:, 0:1] + 1.0)
  dis_tr = lax.rsqrt(dtr_ref[...][:, 0:1] + 1.0)
  ysp_ref[...] = xw[:, :D] * dis_sp
  ytr_ref[...] = xw[:, D:] * dis_tr


def _final_body(ysp_ref, ytr_ref, s_ref, dsp_ref, dtr_ref, bsp_ref, btr_ref,
                wtop_ref, wbot_ref, bf_ref, out_ref):
  dis_sp = lax.rsqrt(dsp_ref[...][:, 0:1] + 1.0)
  dis_tr = lax.rsqrt(dtr_ref[...][:, 0:1] + 1.0)
  s4 = s_ref[...]
  s_sp = jnp.concatenate([s4[0, 0], s4[0, 1]], axis=-1)
  s_tr = jnp.concatenate([s4[1, 0], s4[1, 1]], axis=-1)
  h_s = jnp.maximum((ysp_ref[...] + s_sp) * dis_sp + bsp_ref[...], 0.0)
  h_t = jnp.maximum((ytr_ref[...] + s_tr) * dis_tr + btr_ref[...], 0.0)
  acc = jnp.dot(h_s, wtop_ref[...], preferred_element_type=jnp.float32)
  acc += jnp.dot(h_t, wbot_ref[...], preferred_element_type=jnp.float32)
  out_ref[...] = jnp.maximum(acc + bf_ref[...], 0.0)


def _pad_set(idx, fill):
  return jnp.concatenate(
      [idx, jnp.full((E_PAD - E,), fill, dtype=jnp.int32)])


def kernel(x, sp_ei, tr_ei, W_spa, b_spa, W_tra, b_tra, W_fuse, b_fuse):
  sp_src = sp_ei[0].astype(jnp.int32)
  sp_dst = sp_ei[1].astype(jnp.int32)
  tr_src = tr_ei[0].astype(jnp.int32)
  tr_dst = tr_ei[1].astype(jnp.int32)

  dst_flat = jnp.stack([_pad_set(sp_dst, N), _pad_set(tr_dst, N)])
  dst_deg = dst_flat.reshape(2 * NS, NBLK, B)
  dst_all = dst_flat.reshape(2 * NS * (NBLK // SBLK), SBLK, B)
  gidx = jnp.stack([
      jnp.stack([_pad_set(2 * sp_src, 0), _pad_set(2 * sp_src + 1, 0)]),
      jnp.stack([_pad_set(2 * tr_src, 0), _pad_set(2 * tr_src + 1, 0)]),
  ]).reshape(2 * NC * NS * (NBLK // SBLK), SBLK, B)

  z1 = jnp.zeros((NACC,), jnp.float32)
  z128 = jnp.zeros((NACC, H), jnp.float32)
  ones1 = jnp.ones((B,), jnp.float32)

  deg_flat = _deg_kernel(dst_deg, z1, ones1)
  deg_sp = jnp.broadcast_to(deg_flat[:N, None], (N, 16))
  deg_tr = jnp.broadcast_to(deg_flat[NACC:NACC + N, None], (N, 16))

  W_cat = jnp.concatenate([W_spa, W_tra], axis=1)
  grid = N // BN
  y_sp, y_tr = pl.pallas_call(
      _prep_body,
      grid=(grid,),
      in_specs=[
          pl.BlockSpec((BN, D), lambda i: (i, 0)),
          pl.BlockSpec((D, 2 * D), lambda i: (0, 0)),
          pl.BlockSpec((BN, 16), lambda i: (i, 0)),
          pl.BlockSpec((BN, 16), lambda i: (i, 0)),
      ],
      out_specs=[
          pl.BlockSpec((BN, D), lambda i: (i, 0)),
          pl.BlockSpec((BN, D), lambda i: (i, 0)),
      ],
      out_shape=[
          jax.ShapeDtypeStruct((N, D), jnp.float32),
          jax.ShapeDtypeStruct((N, D), jnp.float32),
      ],
  )(x, W_cat, deg_sp, deg_tr)

  s_flat = _scat_kernel(
      y_sp.reshape(2 * N, H), y_tr.reshape(2 * N, H), gidx, dst_all, z128)
  s4 = s_flat.reshape(2, NC, NACC, H)

  out = pl.pallas_call(
      _final_body,
      grid=(grid,),
      in_specs=[
          pl.BlockSpec((BN, D), lambda i: (i, 0)),
          pl.BlockSpec((BN, D), lambda i: (i, 0)),
          pl.BlockSpec((2, NC, BN, H), lambda i: (0, 0, i, 0)),
          pl.BlockSpec((BN, 16), lambda i: (i, 0)),
          pl.BlockSpec((BN, 16), lambda i: (i, 0)),
          pl.BlockSpec((1, D), lambda i: (0, 0)),
          pl.BlockSpec((1, D), lambda i: (0, 0)),
          pl.BlockSpec((D, D), lambda i: (0, 0)),
          pl.BlockSpec((D, D), lambda i: (0, 0)),
          pl.BlockSpec((1, D), lambda i: (0, 0)),
      ],
      out_specs=pl.BlockSpec((BN, D), lambda i: (i, 0)),
      out_shape=jax.ShapeDtypeStruct((N, D), jnp.float32),
  )(y_sp, y_tr, s4, deg_sp, deg_tr,
    b_spa.reshape(1, D), b_tra.reshape(1, D),
    W_fuse[:D], W_fuse[D:], b_fuse.reshape(1, D))
  return out
